# baseline (speedup 1.0000x reference)
import functools

import jax
import jax.numpy as jnp
from jax import lax
from jax.experimental import pallas as pl
from jax.experimental.pallas import tpu as pltpu
from jax.experimental.pallas import tpu_sc as plsc

N_IDX = 100000
D = 128
C = 128
NW = 32
K = 5
NB = 5
CPW = K * NB
N_CHUNKS = NW * CPW
B_PAD = N_CHUNKS * C

_mesh = plsc.VectorSubcoreMesh(core_axis_name="c", subcore_axis_name="s")


@functools.partial(
    pl.kernel,
    mesh=_mesh,
    out_type=jax.ShapeDtypeStruct((B_PAD, D), jnp.float32),
    scratch_types=(
        [pltpu.VMEM((C,), jnp.int32) for _ in range(K)]
        + [pltpu.VMEM((K, C, D), jnp.float32),
           pltpu.SemaphoreType.DMA,
           pltpu.SemaphoreType.DMA]
    ),
)
def _sc_gather(x_hbm, idx_hbm, out_hbm, *rest):
    idx_bufs, (rows_v, gsem, osem) = rest[:K], rest[K:]
    w = lax.axis_index("s") * 2 + lax.axis_index("c")
    chunk0 = w * CPW

    def wait_outs():
        for _ in range(K):
            pltpu.make_async_copy(rows_v.at[0], out_hbm.at[pl.ds(0, C)],
                                  osem).wait()

    def body(s, carry):
        for b in range(K):
            pltpu.sync_copy(idx_hbm.at[chunk0 + s * K + b], idx_bufs[b])

        @pl.when(s >= 1)
        def _():
            wait_outs()
        gds = [
            pltpu.async_copy(x_hbm.at[idx_bufs[b]], rows_v.at[b], gsem)
            for b in range(K)
        ]
        for gd in gds:
            gd.wait()
        for b in range(K):
            pltpu.async_copy(rows_v.at[b],
                             out_hbm.at[pl.ds((chunk0 + s * K + b) * C, C)],
                             osem)
        return carry

    lax.fori_loop(0, NB, body, 0)
    wait_outs()


def kernel(x, idx):
    idx32 = idx.astype(jnp.int32)
    idx_pad = jnp.zeros((B_PAD,), jnp.int32).at[:N_IDX].set(idx32)
    out = _sc_gather(x, idx_pad.reshape(N_CHUNKS, C))
    return out[:N_IDX]


# speedup vs baseline: 1.3094x; 1.3094x over previous
import functools

import jax
import jax.numpy as jnp
from jax import lax
from jax.experimental import pallas as pl
from jax.experimental.pallas import tpu as pltpu
from jax.experimental.pallas import tpu_sc as plsc

N_IDX = 100000
D = 128
C = 128
NW = 32
K = 5
NB = 5
CPW = K * NB
N_CHUNKS = NW * CPW
B_PAD = N_CHUNKS * C

_mesh = plsc.VectorSubcoreMesh(core_axis_name="c", subcore_axis_name="s")


@functools.partial(
    pl.kernel,
    mesh=_mesh,
    out_type=jax.ShapeDtypeStruct((B_PAD, D), jnp.float32),
    scratch_types=(
        [pltpu.VMEM((C,), jnp.int32) for _ in range(K)]
        + [pltpu.VMEM((K, C, D), jnp.float32),
           pltpu.SemaphoreType.DMA,
           pltpu.SemaphoreType.DMA]
    ),
)
def _sc_gather(x_hbm, idx_hbm, out_hbm, *rest):
    idx_bufs, (rows_v, gsem, osem) = rest[:K], rest[K:]
    w = lax.axis_index("s") * 2 + lax.axis_index("c")

    def wait_outs():
        for _ in range(K):
            pltpu.make_async_copy(rows_v.at[0], out_hbm.at[pl.ds(0, C)],
                                  osem).wait()

    def body(s, carry):
        for b in range(K):
            pltpu.sync_copy(idx_hbm.at[(s * K + b) * NW + w], idx_bufs[b])

        @pl.when(s >= 1)
        def _():
            wait_outs()
        gds = [
            pltpu.async_copy(x_hbm.at[idx_bufs[b]], rows_v.at[b], gsem)
            for b in range(K)
        ]
        for gd in gds:
            gd.wait()
        for b in range(K):
            pltpu.async_copy(rows_v.at[b],
                             out_hbm.at[pl.ds(((s * K + b) * NW + w) * C, C)],
                             osem)
        return carry

    lax.fori_loop(0, NB, body, 0)
    wait_outs()


def kernel(x, idx):
    idx32 = idx.astype(jnp.int32)
    idx_pad = jnp.zeros((B_PAD,), jnp.int32).at[:N_IDX].set(idx32)
    out = _sc_gather(x, idx_pad.reshape(N_CHUNKS, C))
    return out[:N_IDX]
